# Initial kernel scaffold; baseline (speedup 1.0000x reference)
import jax
import jax.numpy as jnp
from jax.experimental import pallas as pl


def kernel(idx, token_table, pos_table, W, b):
    raise NotImplementedError("write your pallas kernel here")


# speedup vs baseline: 1.9118x; 1.9118x over previous
import functools

import jax
import jax.numpy as jnp
from jax import lax
from jax.experimental import pallas as pl
from jax.experimental.pallas import tpu as pltpu
from jax.experimental.pallas import tpu_sc as plsc

_LW = 16


def _table_body(tok_ref, pos_ref, w_ref, b_ref, main_ref, last_ref):
    V = tok_ref.shape[0]
    tw = jnp.dot(tok_ref[:], w_ref[:], preferred_element_type=jnp.float32)
    pw = jnp.dot(pos_ref[:], w_ref[:], preferred_element_type=jnp.float32)
    pwb = pw + b_ref[:]
    vals = pwb[:, None, :] + tw[None, :, :]
    main_ref[:] = vals[:, :, : V - 1]
    last_ref[:, :, :1] = vals[:, :, V - 1 :]
    last_ref[:, :, 1:] = jnp.zeros_like(last_ref[:, :, 1:])


def _build_tables(token_table, pos_table, W, b):
    V, E = token_table.shape
    T = pos_table.shape[0]
    return pl.pallas_call(
        _table_body,
        out_shape=[
            jax.ShapeDtypeStruct((T, V, V - 1), jnp.float32),
            jax.ShapeDtypeStruct((T, V, _LW), jnp.float32),
        ],
    )(token_table, pos_table, W, b.reshape(1, V))


def _make_gather(BT, V, T):
    NC, NS, L = 2, 16, 16
    NW = NC * NS
    assert BT % NW == 0
    b_per_w = BT // NW
    CHUNK = 512
    assert b_per_w % CHUNK == 0
    NCH = b_per_w // CHUNK
    VM = V - 1
    mesh = plsc.VectorSubcoreMesh(core_axis_name="c", subcore_axis_name="s")

    @functools.partial(
        pl.kernel,
        mesh=mesh,
        compiler_params=pltpu.CompilerParams(
            use_tc_tiling_on_sc=False, needs_layout_passes=False
        ),
        out_type=jax.ShapeDtypeStruct((BT, V), jnp.float32),
        scratch_types=[
            pltpu.VMEM((CHUNK,), jnp.int32),
            pltpu.VMEM((CHUNK,), jnp.int32),
            pltpu.VMEM((CHUNK, VM), jnp.float32),
            pltpu.VMEM((CHUNK, VM), jnp.float32),
            pltpu.VMEM((CHUNK, _LW), jnp.float32),
            pltpu.VMEM((CHUNK, _LW), jnp.float32),
            pltpu.VMEM((CHUNK, 1), jnp.float32),
            pltpu.VMEM((CHUNK, 1), jnp.float32),
            pltpu.SemaphoreType.DMA,
            pltpu.SemaphoreType.DMA,
            pltpu.SemaphoreType.DMA,
            pltpu.SemaphoreType.DMA,
            pltpu.SemaphoreType.DMA,
            pltpu.SemaphoreType.DMA,
            pltpu.SemaphoreType.DMA,
            pltpu.SemaphoreType.DMA,
        ],
    )
    def gather_k(
        tmain_hbm, tlast_hbm, idx_hbm, out_hbm,
        idxa, idxb, buf0, buf1, lbuf0, lbuf1, lcol0, lcol1,
        gm0, gm1, gl0, gl1, sm0, sm1, sl0, sl1,
    ):
        wid = lax.axis_index("s") * NC + lax.axis_index("c")
        base = wid * b_per_w

        offs = (lax.iota(jnp.int32, L) % T) * V
        lane = lax.iota(jnp.int32, L)
        zeros = lane * 0
        idxbufs = (idxa, idxb)
        bufs = (buf0, buf1)
        lbufs = (lbuf0, lbuf1)
        lcols = (lcol0, lcol1)
        gmsems = (gm0, gm1)
        glsems = (gl0, gl1)
        smsems = (sm0, sm1)
        slsems = (sl0, sl1)

        def load_idx(c):
            ib = idxbufs[c & 1]
            pltpu.sync_copy(idx_hbm.at[pl.ds(base + c * CHUNK, CHUNK)], ib)

            def add_offs(j, _):
                sl = pl.ds(j * L, L)
                ib[sl] = ib[sl] + offs
                return 0

            lax.fori_loop(0, CHUNK // L, add_offs, 0)

        def start_gathers(c):
            s = c & 1
            ib = idxbufs[s]
            dm = pltpu.async_copy(tmain_hbm.at[ib], bufs[s], gmsems[s])
            dl = pltpu.async_copy(tlast_hbm.at[ib], lbufs[s], glsems[s])
            return dm, dl

        def patch_lcol(s):
            lb, lc = lbufs[s], lcols[s]

            def fix(j, _):
                rvec = j * L + lane
                vals = plsc.load_gather(lb, [rvec, zeros])
                plsc.store_scatter(lc, [rvec, zeros], vals)
                return 0

            lax.fori_loop(0, CHUNK // L, fix, 0)

        gd = {}
        sd = {}
        load_idx(0)
        gd[0] = start_gathers(0)
        for c in range(NCH):
            bsel = c & 1
            nsel = (c + 1) & 1
            if c + 1 < NCH:
                load_idx(c + 1)
                if c >= 1:
                    for d in sd[c - 1]:
                        d.wait()
                gd[c + 1] = start_gathers(c + 1)
            for d in gd[c]:
                d.wait()
            patch_lcol(bsel)
            rows = pl.ds(base + c * CHUNK, CHUNK)
            sd[c] = (
                pltpu.async_copy(
                    bufs[bsel], out_hbm.at[rows, pl.ds(0, VM)], smsems[bsel]
                ),
                pltpu.async_copy(
                    lcols[bsel], out_hbm.at[rows, pl.ds(VM, 1)], slsems[bsel]
                ),
            )
        if NCH >= 2:
            for d in sd[NCH - 2]:
                d.wait()
        for d in sd[NCH - 1]:
            d.wait()

    return gather_k


def kernel(idx, token_table, pos_table, W, b):
    B, T = idx.shape
    V = token_table.shape[0]
    BT = B * T

    tmain, tlast = _build_tables(token_table, pos_table, W, b)

    gather_k = _make_gather(BT, V, T)
    idx_flat = idx.reshape(BT).astype(jnp.int32)
    out = gather_k(tmain.reshape(T * V, V - 1), tlast.reshape(T * V, _LW), idx_flat)
    return out.reshape(B, T, V)


# speedup vs baseline: 2.9918x; 1.5650x over previous
import functools

import jax
import jax.numpy as jnp
from jax import lax
from jax.experimental import pallas as pl
from jax.experimental.pallas import tpu as pltpu
from jax.experimental.pallas import tpu_sc as plsc

_ROW = 128


def _table_body(tok_ref, pos_ref, w_ref, b_ref, out_ref):
    V = tok_ref.shape[0]
    tw = jnp.dot(tok_ref[:], w_ref[:], preferred_element_type=jnp.float32)
    pw = jnp.dot(pos_ref[:], w_ref[:], preferred_element_type=jnp.float32)
    pwb = pw + b_ref[:]
    out_ref[:, :, :V] = pwb[:, None, :] + tw[None, :, :]
    out_ref[:, :, V:] = jnp.zeros_like(out_ref[:, :, V:])


def _build_table(token_table, pos_table, W, b):
    V, E = token_table.shape
    T = pos_table.shape[0]
    return pl.pallas_call(
        _table_body,
        out_shape=jax.ShapeDtypeStruct((T, V, _ROW), jnp.float32),
    )(token_table, pos_table, W, b.reshape(1, V))


def _make_gather(BT, V, T):
    NC, NS, L = 2, 16, 16
    NW = NC * NS
    assert BT % NW == 0
    b_per_w = BT // NW
    CHUNK = 256
    assert b_per_w % CHUNK == 0
    NCH = b_per_w // CHUNK
    mesh = plsc.VectorSubcoreMesh(core_axis_name="c", subcore_axis_name="s")

    @functools.partial(
        pl.kernel,
        mesh=mesh,
        out_type=jax.ShapeDtypeStruct((BT, _ROW), jnp.float32),
        scratch_types=[
            pltpu.VMEM((CHUNK,), jnp.int32),
            pltpu.VMEM((CHUNK,), jnp.int32),
            pltpu.VMEM((CHUNK, _ROW), jnp.float32),
            pltpu.VMEM((CHUNK, _ROW), jnp.float32),
            pltpu.SemaphoreType.DMA,
            pltpu.SemaphoreType.DMA,
            pltpu.SemaphoreType.DMA,
            pltpu.SemaphoreType.DMA,
        ],
    )
    def gather_k(table_hbm, idx_hbm, out_hbm, idxa, idxb, buf0, buf1, g0, g1, s0, s1):
        wid = lax.axis_index("s") * NC + lax.axis_index("c")
        base = wid * b_per_w

        offs = (lax.iota(jnp.int32, L) % T) * V
        idxbufs = (idxa, idxb)
        bufs = (buf0, buf1)
        gsems = (g0, g1)
        ssems = (s0, s1)

        def load_idx(c):
            ib = idxbufs[c & 1]
            pltpu.sync_copy(idx_hbm.at[pl.ds(base + c * CHUNK, CHUNK)], ib)

            def add_offs(j, _):
                sl = pl.ds(j * L, L)
                ib[sl] = ib[sl] + offs
                return 0

            lax.fori_loop(0, CHUNK // L, add_offs, 0)

        gd = {}
        sd = {}
        load_idx(0)
        gd[0] = pltpu.async_copy(table_hbm.at[idxbufs[0]], bufs[0], gsems[0])
        for c in range(NCH):
            bsel = c & 1
            nsel = (c + 1) & 1
            if c + 1 < NCH:
                load_idx(c + 1)
                if c >= 1:
                    sd[c - 1].wait()
                gd[c + 1] = pltpu.async_copy(
                    table_hbm.at[idxbufs[nsel]], bufs[nsel], gsems[nsel]
                )
            gd[c].wait()
            sd[c] = pltpu.async_copy(
                bufs[bsel], out_hbm.at[pl.ds(base + c * CHUNK, CHUNK)], ssems[bsel]
            )
        if NCH >= 2:
            sd[NCH - 2].wait()
        sd[NCH - 1].wait()

    return gather_k


def kernel(idx, token_table, pos_table, W, b):
    B, T = idx.shape
    V = token_table.shape[0]
    BT = B * T

    table = _build_table(token_table, pos_table, W, b).reshape(T * V, _ROW)

    gather_k = _make_gather(BT, V, T)
    idx_flat = idx.reshape(BT).astype(jnp.int32)
    out = gather_k(table, idx_flat)
    return out.reshape(B, T, _ROW)[:, :, :V]


# speedup vs baseline: 4.8459x; 1.6197x over previous
import functools

import jax
import jax.numpy as jnp
from jax import lax
from jax.experimental import pallas as pl
from jax.experimental.pallas import tpu as pltpu
from jax.experimental.pallas import tpu_sc as plsc

_ROW = 128


def _table_body(tok_ref, pos_ref, w_ref, b_ref, out_ref):
    V = tok_ref.shape[0]
    tw = jnp.dot(tok_ref[:], w_ref[:], preferred_element_type=jnp.float32)
    pw = jnp.dot(pos_ref[:], w_ref[:], preferred_element_type=jnp.float32)
    pwb = pw + b_ref[:]
    out_ref[:, :, :V] = pwb[:, None, :] + tw[None, :, :]
    out_ref[:, :, V:] = jnp.zeros_like(out_ref[:, :, V:])


def _build_table(token_table, pos_table, W, b):
    V, E = token_table.shape
    T = pos_table.shape[0]
    return pl.pallas_call(
        _table_body,
        out_shape=jax.ShapeDtypeStruct((T, V, _ROW), jnp.float32),
    )(token_table, pos_table, W, b.reshape(1, V))


def _make_gather(BT, V, T):
    NC, NS, L = 2, 16, 16
    NW = NC * NS
    assert BT % NW == 0
    b_per_w = BT // NW
    CHUNK = 256
    assert b_per_w % CHUNK == 0
    NCH = b_per_w // CHUNK
    mesh = plsc.VectorSubcoreMesh(core_axis_name="c", subcore_axis_name="s")

    @functools.partial(
        pl.kernel,
        mesh=mesh,
        out_type=jax.ShapeDtypeStruct((BT, _ROW), jnp.float32),
        scratch_types=[
            pltpu.VMEM_SHARED((T * V, _ROW), jnp.float32),
            pltpu.VMEM((CHUNK,), jnp.int32),
            pltpu.VMEM((CHUNK,), jnp.int32),
            pltpu.VMEM((CHUNK, _ROW), jnp.float32),
            pltpu.VMEM((CHUNK, _ROW), jnp.float32),
            pltpu.SemaphoreType.DMA,
            pltpu.SemaphoreType.DMA,
            pltpu.SemaphoreType.DMA,
            pltpu.SemaphoreType.DMA,
        ],
    )
    def gather_k(
        table_hbm, idx_hbm, out_hbm, table_sh, idxa, idxb, buf0, buf1, g0, g1, s0, s1
    ):
        sid = lax.axis_index("s")
        wid = sid * NC + lax.axis_index("c")
        base = wid * b_per_w

        @pl.when(sid == 0)
        def _():
            pltpu.sync_copy(table_hbm, table_sh)

        plsc.subcore_barrier()

        offs = (lax.iota(jnp.int32, L) % T) * V
        idxbufs = (idxa, idxb)
        bufs = (buf0, buf1)
        gsems = (g0, g1)
        ssems = (s0, s1)

        def load_idx(c):
            ib = idxbufs[c & 1]
            pltpu.sync_copy(idx_hbm.at[pl.ds(base + c * CHUNK, CHUNK)], ib)

            def add_offs(j, _):
                sl = pl.ds(j * L, L)
                ib[sl] = ib[sl] + offs
                return 0

            lax.fori_loop(0, CHUNK // L, add_offs, 0)

        gd = {}
        sd = {}
        load_idx(0)
        gd[0] = pltpu.async_copy(table_sh.at[idxbufs[0]], bufs[0], gsems[0])
        for c in range(NCH):
            bsel = c & 1
            nsel = (c + 1) & 1
            if c + 1 < NCH:
                load_idx(c + 1)
                if c >= 1:
                    sd[c - 1].wait()
                gd[c + 1] = pltpu.async_copy(
                    table_sh.at[idxbufs[nsel]], bufs[nsel], gsems[nsel]
                )
            gd[c].wait()
            sd[c] = pltpu.async_copy(
                bufs[bsel], out_hbm.at[pl.ds(base + c * CHUNK, CHUNK)], ssems[bsel]
            )
        if NCH >= 2:
            sd[NCH - 2].wait()
        sd[NCH - 1].wait()

    return gather_k


def kernel(idx, token_table, pos_table, W, b):
    B, T = idx.shape
    V = token_table.shape[0]
    BT = B * T

    table = _build_table(token_table, pos_table, W, b).reshape(T * V, _ROW)

    gather_k = _make_gather(BT, V, T)
    idx_flat = idx.reshape(BT).astype(jnp.int32)
    out = gather_k(table, idx_flat)
    return out.reshape(B, T, _ROW)[:, :, :V]
